# baseline (speedup 1.0000x reference)
import functools

import jax
import jax.numpy as jnp
from jax import lax
from jax.experimental import pallas as pl
from jax.experimental.pallas import tpu as pltpu
from jax.experimental.pallas import tpu_sc as plsc

N_EMBD = 64
PADW = 128
SEQ = 200
BATCH = 4096
N_TOK = 100000

_INFO = plsc.get_sparse_core_info()
NC, NS, L = _INFO.num_cores, _INFO.num_subcores, _INFO.num_lanes
NW = NC * NS

BBLK = BATCH // NW
RING = 4

_mesh = plsc.VectorSubcoreMesh(core_axis_name="c", subcore_axis_name="s")


@functools.partial(
    pl.kernel,
    mesh=_mesh,
    out_type=jax.ShapeDtypeStruct((SEQ, N_EMBD, BATCH), jnp.float32),
    scratch_types=[
        pltpu.VMEM((SEQ, BBLK), jnp.int32),
        pltpu.VMEM((RING, PADW), jnp.float32),
        pltpu.VMEM((RING, BBLK, PADW), jnp.float32),
        pltpu.VMEM((RING, N_EMBD, BBLK), jnp.float32),
    ]
    + [pltpu.SemaphoreType.DMA] * RING
    + [pltpu.SemaphoreType.DMA] * RING
    + [pltpu.SemaphoreType.DMA] * RING,
    compiler_params=pltpu.CompilerParams(needs_layout_passes=False),
)
def _emb_lookup(
    xt_hbm, tok_hbm, pos_hbm, out_hbm, idx_v, posr_v, rows_v, outs_v, *sems
):
    gsem = sems[:RING]
    osem = sems[RING : 2 * RING]
    psem = sems[2 * RING :]
    wid = lax.axis_index("s") * NC + lax.axis_index("c")
    col0 = wid * BBLK

    pltpu.sync_copy(xt_hbm.at[:, pl.ds(col0, BBLK)], idx_v)

    def issue_gather(s, slot):
        pltpu.async_copy(tok_hbm.at[idx_v.at[s]], rows_v.at[slot], gsem[slot])
        pltpu.async_copy(pos_hbm.at[s], posr_v.at[slot, pl.ds(8, N_EMBD)], psem[slot])

    issue_gather(0, 0)
    issue_gather(1, 1)

    lane_ids = [lax.iota(jnp.int32, L) + j * L for j in range(BBLK // L)]

    def group_body(gg, carry):
        for b in range(RING):
            s = gg * RING + b

            @pl.when(s + 2 < SEQ)
            def _():
                issue_gather(s + 2, (b + 2) % RING)

            pltpu.make_async_copy(
                tok_hbm.at[idx_v.at[0]], rows_v.at[b], gsem[b]
            ).wait()
            pltpu.make_async_copy(
                pos_hbm.at[0], posr_v.at[b, pl.ds(8, N_EMBD)], psem[b]
            ).wait()

            @pl.when(s >= RING)
            def _():
                pltpu.make_async_copy(
                    outs_v.at[b],
                    out_hbm.at[0, :, pl.ds(col0, BBLK)],
                    osem[b],
                ).wait()

            @plsc.parallel_loop(0, N_EMBD, step=1, unroll=16)
            def _(e):
                ecol = jnp.full((L,), 0, jnp.int32) + e
                ps = plsc.load_gather(posr_v.at[b], [ecol + 8])
                for j in range(BBLK // L):
                    vals = plsc.load_gather(rows_v.at[b], [lane_ids[j], ecol])
                    outs_v[b, e, pl.ds(j * L, L)] = vals + ps

            pltpu.async_copy(
                outs_v.at[b],
                out_hbm.at[s, :, pl.ds(col0, BBLK)],
                osem[b],
            )
        return carry

    lax.fori_loop(0, SEQ // RING, group_body, 0)

    for b in range(RING):
        pltpu.make_async_copy(
            outs_v.at[b],
            out_hbm.at[0, :, pl.ds(col0, BBLK)],
            osem[b],
        ).wait()


def kernel(x, tok_emb, pos_emb):
    xt = x.T.astype(jnp.int32)
    tok_pad = jnp.pad(tok_emb, ((0, 0), (0, PADW - N_EMBD)))
    out = _emb_lookup(xt, tok_pad, pos_emb)
    return jnp.transpose(out, (2, 0, 1))


# speedup vs baseline: 1.0298x; 1.0298x over previous
import functools

import jax
import jax.numpy as jnp
from jax import lax
from jax.experimental import pallas as pl
from jax.experimental.pallas import tpu as pltpu
from jax.experimental.pallas import tpu_sc as plsc

N_EMBD = 64
SEQ = 200
BATCH = 4096
N_ROWS = BATCH * SEQ

_INFO = plsc.get_sparse_core_info()
NC, NS, L = _INFO.num_cores, _INFO.num_subcores, _INFO.num_lanes
NW = NC * NS

SEQ_PER_WORKER = BATCH // NW
CHUNK_SEQS = 2
CHUNK_ROWS = CHUNK_SEQS * SEQ
CHUNKS = SEQ_PER_WORKER // CHUNK_SEQS
ROWS_PER_WORKER = SEQ_PER_WORKER * SEQ
RING = 4

_mesh = plsc.VectorSubcoreMesh(core_axis_name="c", subcore_axis_name="s")


@functools.partial(
    pl.kernel,
    mesh=_mesh,
    out_type=jax.ShapeDtypeStruct((N_ROWS, N_EMBD), jnp.float32),
    scratch_types=[
        pltpu.VMEM((SEQ, N_EMBD), jnp.float32),
        pltpu.VMEM((RING, CHUNK_ROWS), jnp.int32),
        pltpu.VMEM((RING, CHUNK_ROWS, N_EMBD), jnp.float32),
    ]
    + [pltpu.SemaphoreType.DMA] * RING
    + [pltpu.SemaphoreType.DMA] * RING,
    compiler_params=pltpu.CompilerParams(use_tc_tiling_on_sc=False),
)
def _emb_lookup(idx_hbm, tok_hbm, pos_hbm, out_hbm, pos_v, idx_v, rows_v, *sems):
    gsem = sems[:RING]
    osem = sems[RING:]
    wid = lax.axis_index("s") * NC + lax.axis_index("c")
    base_row = wid * ROWS_PER_WORKER
    pltpu.sync_copy(pos_hbm, pos_v)

    def issue_gather(h, slot):
        row0 = base_row + h * CHUNK_ROWS
        pltpu.sync_copy(idx_hbm.at[pl.ds(row0, CHUNK_ROWS)], idx_v.at[slot])
        pltpu.async_copy(tok_hbm.at[idx_v.at[slot]], rows_v.at[slot], gsem[slot])

    issue_gather(0, 0)
    issue_gather(1, 1)

    def group_body(gg, carry):
        for b in range(RING):
            g = gg * RING + b
            hb = (b + 2) % RING

            @pl.when(g + 2 < CHUNKS)
            def _():
                @pl.when(g + 2 >= RING)
                def _():
                    pltpu.make_async_copy(
                        rows_v.at[hb],
                        out_hbm.at[pl.ds(base_row, CHUNK_ROWS)],
                        osem[hb],
                    ).wait()
                issue_gather(g + 2, hb)

            pltpu.make_async_copy(
                tok_hbm.at[idx_v.at[b]], rows_v.at[b], gsem[b]
            ).wait()

            def row_body(r, carry2):
                for c in range(N_EMBD // L):
                    col = pl.ds(c * L, L)
                    pvec = pos_v[r, col]
                    for s in range(CHUNK_SEQS):
                        rr = s * SEQ + r
                        rows_v[b, rr, col] = rows_v[b, rr, col] + pvec
                return carry2

            lax.fori_loop(0, SEQ, row_body, 0)

            pltpu.async_copy(
                rows_v.at[b],
                out_hbm.at[pl.ds(base_row + g * CHUNK_ROWS, CHUNK_ROWS)],
                osem[b],
            )
        return carry

    lax.fori_loop(0, CHUNKS // RING, group_body, 0)

    for b in range(RING):
        pltpu.make_async_copy(
            rows_v.at[b],
            out_hbm.at[pl.ds(base_row, CHUNK_ROWS)],
            osem[b],
        ).wait()


def kernel(x, tok_emb, pos_emb):
    idx = x.reshape(-1).astype(jnp.int32)
    out = _emb_lookup(idx, tok_emb, pos_emb)
    return out.reshape(x.shape[0], x.shape[1], N_EMBD)
